# Initial kernel scaffold; baseline (speedup 1.0000x reference)
import jax
import jax.numpy as jnp
from jax.experimental import pallas as pl


def kernel(x, fc_weight, bias, embedding_weight):
    raise NotImplementedError("write your pallas kernel here")


# speedup vs baseline: 16.5108x; 16.5108x over previous
import functools

import jax
import jax.numpy as jnp
from jax import lax
from jax.experimental import pallas as pl
from jax.experimental.pallas import tpu as pltpu
from jax.experimental.pallas import tpu_sc as plsc

_NC = 2
_NS = 16
_NW = _NC * _NS
_L = 16
_PAD = 16


def _combine_body(bias_ref, fc_ref, emb_ref, out_ref):
    e = emb_ref[...]
    c = (fc_ref[...] - 0.5 * jnp.sum(e * e, axis=1, keepdims=True)
         + bias_ref[0, 0])
    out_ref[...] = jnp.broadcast_to(c, (c.shape[0], _PAD))


def _make_combined_table(fc_weight, embedding_weight, bias, n_fields):
    v, d = embedding_weight.shape
    rb = 1000
    assert v % rb == 0
    bias_over_f = (bias.astype(jnp.float32) / n_fields).reshape(1, 1)
    return pl.pallas_call(
        _combine_body,
        grid=(v // rb,),
        in_specs=[
            pl.BlockSpec(memory_space=pltpu.SMEM),
            pl.BlockSpec((rb, 1), lambda i: (i, 0)),
            pl.BlockSpec((rb, d), lambda i: (i, 0)),
        ],
        out_specs=pl.BlockSpec((rb, _PAD), lambda i: (i, 0)),
        out_shape=jax.ShapeDtypeStruct((v, _PAD), jnp.float32),
    )(bias_over_f, fc_weight, embedding_weight)


def _fm_sc_kernel(x3, c_tab, emb, *, batch, n_fields, d):
    bpw = batch // _NW
    nch = bpw // 128
    mesh = plsc.VectorSubcoreMesh(
        core_axis_name="c", subcore_axis_name="s",
        num_cores=_NC, num_subcores=_NS,
    )

    @functools.partial(
        pl.kernel,
        out_type=(
            jax.ShapeDtypeStruct((batch, d), jnp.float32),
            jax.ShapeDtypeStruct((batch, _PAD), jnp.float32),
        ),
        mesh=mesh,
        compiler_params=pltpu.CompilerParams(use_tc_tiling_on_sc=False),
        scratch_types=[
            pltpu.VMEM((n_fields, nch, 128), jnp.int32),
            pltpu.VMEM((bpw, d), jnp.float32),
            pltpu.VMEM((bpw, _PAD), jnp.float32),
            pltpu.SemaphoreType.DMA,
        ],
    )
    def body(x3_hbm, c_hbm, emb_hbm, s_hbm, zacc_hbm, idx_v, s_v, acc_v, sem):
        cid = lax.axis_index("c")
        sid = lax.axis_index("s")
        wid = sid * _NC + cid
        base = wid * bpw
        cb = wid * nch

        idx_cps = [
            pltpu.async_copy(x3_hbm.at[f, pl.ds(cb, nch), :], idx_v.at[f], sem)
            for f in range(n_fields)
        ]

        zeros = jnp.zeros((_L,), jnp.float32)

        def zero_row(i, carry):
            for j in range(d // _L):
                s_v[i, pl.ds(j * _L, _L)] = zeros
            return carry

        lax.fori_loop(0, bpw, zero_row, 0)

        def zero_acc(i, carry):
            acc_v[i, pl.ds(0, _L)] = zeros
            return carry

        lax.fori_loop(0, bpw, zero_acc, 0)
        for cp in idx_cps:
            cp.wait()

        def gather_field(f, carry):
            cps = []
            for ch in range(nch):
                cps.append(pltpu.async_copy(
                    emb_hbm.at[idx_v.at[f, ch]],
                    s_v.at[pl.ds(ch * 128, 128)],
                    sem, add=True))
                cps.append(pltpu.async_copy(
                    c_hbm.at[idx_v.at[f, ch]],
                    acc_v.at[pl.ds(ch * 128, 128)],
                    sem, add=True))
            for cp in cps:
                cp.wait()
            return carry

        lax.fori_loop(0, n_fields, gather_field, 0)

        pltpu.sync_copy(s_v, s_hbm.at[pl.ds(base, bpw), :])
        pltpu.sync_copy(acc_v, zacc_hbm.at[pl.ds(base, bpw), :])

    return body(x3, c_tab, emb)


def _epilogue_body(s_ref, zacc_ref, out_ref):
    s = s_ref[...]
    z = zacc_ref[:, :1] + 0.5 * jnp.sum(s * s, axis=1, keepdims=True)
    out_ref[...] = 1.0 / (1.0 + jnp.exp(-z))


def _epilogue(s, zacc, *, batch, d):
    rb = 2048
    assert batch % rb == 0
    out = pl.pallas_call(
        _epilogue_body,
        grid=(batch // rb,),
        in_specs=[
            pl.BlockSpec((rb, d), lambda i: (i, 0)),
            pl.BlockSpec((rb, _PAD), lambda i: (i, 0)),
        ],
        out_specs=pl.BlockSpec((rb, 1), lambda i: (i, 0)),
        out_shape=jax.ShapeDtypeStruct((batch, 1), jnp.float32),
    )(s, zacc)
    return out.reshape(batch)


def kernel(x, fc_weight, bias, embedding_weight):
    batch, n_fields = x.shape
    v, d = embedding_weight.shape
    assert batch % (_NW * 128) == 0 and d % _L == 0

    c_tab = _make_combined_table(fc_weight, embedding_weight, bias, n_fields)
    x3 = x.astype(jnp.int32).T.reshape(n_fields, batch // 128, 128)
    s, zacc = _fm_sc_kernel(x3, c_tab, embedding_weight,
                            batch=batch, n_fields=n_fields, d=d)
    return _epilogue(s, zacc, batch=batch, d=d)
